# baseline (speedup 1.0000x reference)
import functools

import jax
import jax.numpy as jnp
from jax import lax
from jax.experimental import pallas as pl
from jax.experimental.pallas import tpu as pltpu
from jax.experimental.pallas import tpu_sc as plsc

NC = 2
NS = 16
L = 16
NW = NC * NS

B = 16384
D = 32
V = 1000000
BPW = B // NW
SE = 4
NSUB = BPW // SE
HT = D // 4

_mesh = plsc.VectorSubcoreMesh(core_axis_name="c", subcore_axis_name="s")


@functools.partial(
    pl.kernel,
    out_type=jax.ShapeDtypeStruct((B,), jnp.float32),
    mesh=_mesh,
    compiler_params=pltpu.CompilerParams(needs_layout_passes=False,
                                         use_tc_tiling_on_sc=True),
    scratch_types=[
        pltpu.VMEM((BPW + L,), jnp.int32),
        pltpu.VMEM((BPW + L,), jnp.int32),
        pltpu.VMEM((2, SE, 4, 8, 128), jnp.float32),
        pltpu.VMEM((2, SE, 4, 8, 128), jnp.float32),
        pltpu.VMEM((HT, L), jnp.float32),
        pltpu.VMEM((L,), jnp.float32),
        pltpu.VMEM((BPW + L,), jnp.float32),
        pltpu.SemaphoreType.DMA,
        pltpu.SemaphoreType.DMA,
    ],
)
def _mf_sc_kernel(uidx_hbm, iidx_hbm, utab3_hbm, itab3_hbm, out_hbm,
                  uidx_v, iidx_v, ublk_v, vblk_v, ucol_v, fold_v, out_v,
                  usem, vsem):
    sid = lax.axis_index("s")
    cid = lax.axis_index("c")
    wid = sid * NC + cid
    base = wid * BPW

    pltpu.sync_copy(uidx_hbm.at[pl.ds(base, BPW)], uidx_v.at[pl.ds(0, BPW)])
    pltpu.sync_copy(iidx_hbm.at[pl.ds(base, BPW)], iidx_v.at[pl.ds(0, BPW)])

    lane = lax.iota(jnp.int32, L)
    lo4 = lax.bitwise_and(lane, 3)
    quad = lax.shift_right_logical(lane, 2)
    outmask = quad == 0

    def issue(tab_hbm, idx_ref, blk, s, par, sem):
        def body(e, c):
            vec = idx_ref[pl.ds(s * SE + e, L)]
            o0 = pl.multiple_of(lax.bitwise_and(vec[0], jnp.int32(~127)), 128)
            pltpu.async_copy(tab_hbm.at[:, :, pl.ds(o0, 128)],
                             blk.at[par, e], sem)
            return c
        lax.fori_loop(0, SE, body, 0)

    udrain = pltpu.make_async_copy(utab3_hbm.at[:, :, pl.ds(0, 128)],
                                   ublk_v.at[0, 0], usem)
    vdrain = pltpu.make_async_copy(itab3_hbm.at[:, :, pl.ds(0, 128)],
                                   vblk_v.at[0, 0], vsem)

    issue(utab3_hbm, uidx_v, ublk_v, 0, 0, usem)
    issue(itab3_hbm, iidx_v, vblk_v, 0, 0, vsem)

    def sub_body(s, carry):
        e0 = s * SE
        par = lax.bitwise_and(s, 1)
        parn = lax.bitwise_and(s + 1, 1)
        parv = jnp.zeros((L,), jnp.int32) + par

        for _ in range(SE):
            udrain.wait()

        @pl.when(s + 1 < NSUB)
        def _():
            issue(utab3_hbm, uidx_v, ublk_v, s + 1, parn, usem)

        up = lax.bitwise_and(plsc.load_gather(uidx_v, [e0 + lo4]), jnp.int32(127))
        for t in range(HT):
            j0 = 4 * t
            fb = jnp.full((L,), j0 // 8, jnp.int32)
            fr = jnp.full((L,), j0 % 8, jnp.int32) + quad
            ucol_v[t, :] = plsc.load_gather(ublk_v, [parv, lo4, fb, fr, up])

        for _ in range(SE):
            vdrain.wait()

        @pl.when(s + 1 < NSUB)
        def _():
            issue(itab3_hbm, iidx_v, vblk_v, s + 1, parn, vsem)

        ip = lax.bitwise_and(plsc.load_gather(iidx_v, [e0 + lo4]), jnp.int32(127))
        acc = jnp.zeros((L,), jnp.float32)
        for t in range(HT):
            j0 = 4 * t
            fb = jnp.full((L,), j0 // 8, jnp.int32)
            fr = jnp.full((L,), j0 % 8, jnp.int32) + quad
            vj = plsc.load_gather(vblk_v, [parv, lo4, fb, fr, ip])
            acc = acc + ucol_v[t, :] * vj

        fold_v[...] = acc
        acc2 = acc + plsc.load_gather(fold_v, [lax.bitwise_xor(lane, 8)])
        fold_v[...] = acc2
        tot = acc2 + plsc.load_gather(fold_v, [lax.bitwise_xor(lane, 4)])
        plsc.store_compressed(out_v.at[pl.ds(e0, L)], tot, mask=outmask)
        return carry

    lax.fori_loop(0, NSUB, sub_body, 0)
    pltpu.sync_copy(out_v.at[pl.ds(0, BPW)], out_hbm.at[pl.ds(base, BPW)])


def kernel(user_idx, item_idx, user_table, item_table):
    ut3 = user_table.T.reshape(4, 8, V)
    it3 = item_table.T.reshape(4, 8, V)
    return _mf_sc_kernel(user_idx.astype(jnp.int32), item_idx.astype(jnp.int32),
                         ut3, it3)


# speedup vs baseline: 1.1019x; 1.1019x over previous
import functools

import jax
import jax.numpy as jnp
from jax import lax
from jax.experimental import pallas as pl
from jax.experimental.pallas import tpu as pltpu
from jax.experimental.pallas import tpu_sc as plsc

NC = 2
NS = 16
L = 16
NW = NC * NS

B = 16384
D = 32
V = 1000000
BPW = B // NW
SE = 4
NSUB = BPW // SE
HT = D // 4

_mesh = plsc.VectorSubcoreMesh(core_axis_name="c", subcore_axis_name="s")


@functools.partial(
    pl.kernel,
    out_type=jax.ShapeDtypeStruct((B,), jnp.float32),
    mesh=_mesh,
    compiler_params=pltpu.CompilerParams(needs_layout_passes=False,
                                         use_tc_tiling_on_sc=True),
    scratch_types=[
        pltpu.VMEM((BPW + L,), jnp.int32),
        pltpu.VMEM((BPW + L,), jnp.int32),
        pltpu.VMEM((2, SE, 4, 8, 128), jnp.float32),
        pltpu.VMEM((2, SE, 4, 8, 128), jnp.float32),
        pltpu.VMEM((L,), jnp.float32),
        pltpu.VMEM((BPW + L,), jnp.float32),
        pltpu.SemaphoreType.DMA,
        pltpu.SemaphoreType.DMA,
        pltpu.SemaphoreType.DMA,
        pltpu.SemaphoreType.DMA,
    ],
)
def _mf_sc_kernel(uidx_hbm, iidx_hbm, utab3_hbm, itab3_hbm, out_hbm,
                  uidx_v, iidx_v, ublk_v, vblk_v, fold_v, out_v,
                  usem0, usem1, vsem0, vsem1):
    sid = lax.axis_index("s")
    cid = lax.axis_index("c")
    wid = sid * NC + cid
    base = wid * BPW

    pltpu.sync_copy(uidx_hbm.at[pl.ds(base, BPW)], uidx_v.at[pl.ds(0, BPW)])
    pltpu.sync_copy(iidx_hbm.at[pl.ds(base, BPW)], iidx_v.at[pl.ds(0, BPW)])

    lane = lax.iota(jnp.int32, L)
    lo4 = lax.bitwise_and(lane, 3)
    quad = lax.shift_right_logical(lane, 2)
    outmask = quad == 0

    def issue(tab_hbm, idx_ref, blk, s, par, sem):
        def body(e, c):
            vec = idx_ref[pl.ds(s * SE + e, L)]
            o0 = pl.multiple_of(lax.bitwise_and(vec[0], jnp.int32(~127)), 128)
            pltpu.async_copy(tab_hbm.at[:, :, pl.ds(o0, 128)],
                             blk.at[par, e], sem)
            return c
        lax.fori_loop(0, SE, body, 0)

    def drain(tab_hbm, blk, sem):
        tmpl = pltpu.make_async_copy(tab_hbm.at[:, :, pl.ds(0, 128)],
                                     blk.at[0, 0], sem)
        for _ in range(SE):
            tmpl.wait()

    issue(utab3_hbm, uidx_v, ublk_v, 0, 0, usem0)
    issue(itab3_hbm, iidx_v, vblk_v, 0, 0, vsem0)
    issue(utab3_hbm, uidx_v, ublk_v, 1, 1, usem1)
    issue(itab3_hbm, iidx_v, vblk_v, 1, 1, vsem1)

    def sub_body(s, carry):
        e0 = s * SE
        par = lax.bitwise_and(s, 1)
        parv = jnp.zeros((L,), jnp.int32) + par

        @pl.when(par == 0)
        def _():
            drain(utab3_hbm, ublk_v, usem0)
            drain(itab3_hbm, vblk_v, vsem0)

        @pl.when(par == 1)
        def _():
            drain(utab3_hbm, ublk_v, usem1)
            drain(itab3_hbm, vblk_v, vsem1)

        up = lax.bitwise_and(plsc.load_gather(uidx_v, [e0 + lo4]), jnp.int32(127))
        ip = lax.bitwise_and(plsc.load_gather(iidx_v, [e0 + lo4]), jnp.int32(127))
        acc = jnp.zeros((L,), jnp.float32)
        for t in range(HT):
            j0 = 4 * t
            fb = jnp.full((L,), j0 // 8, jnp.int32)
            fr = jnp.full((L,), j0 % 8, jnp.int32) + quad
            uj = plsc.load_gather(ublk_v, [parv, lo4, fb, fr, up])
            vj = plsc.load_gather(vblk_v, [parv, lo4, fb, fr, ip])
            acc = acc + uj * vj

        fold_v[...] = acc
        acc2 = acc + plsc.load_gather(fold_v, [lax.bitwise_xor(lane, 8)])
        fold_v[...] = acc2
        tot = acc2 + plsc.load_gather(fold_v, [lax.bitwise_xor(lane, 4)])
        plsc.store_compressed(out_v.at[pl.ds(e0, L)], tot, mask=outmask)

        @pl.when((s + 2 < NSUB) & (par == 0))
        def _():
            issue(utab3_hbm, uidx_v, ublk_v, s + 2, 0, usem0)
            issue(itab3_hbm, iidx_v, vblk_v, s + 2, 0, vsem0)

        @pl.when((s + 2 < NSUB) & (par == 1))
        def _():
            issue(utab3_hbm, uidx_v, ublk_v, s + 2, 1, usem1)
            issue(itab3_hbm, iidx_v, vblk_v, s + 2, 1, vsem1)
        return carry

    lax.fori_loop(0, NSUB, sub_body, 0)
    pltpu.sync_copy(out_v.at[pl.ds(0, BPW)], out_hbm.at[pl.ds(base, BPW)])


def kernel(user_idx, item_idx, user_table, item_table):
    ut3 = user_table.T.reshape(4, 8, V)
    it3 = item_table.T.reshape(4, 8, V)
    return _mf_sc_kernel(user_idx.astype(jnp.int32), item_idx.astype(jnp.int32),
                         ut3, it3)


# speedup vs baseline: 1.2018x; 1.0906x over previous
import functools

import jax
import jax.numpy as jnp
from jax import lax
from jax.experimental import pallas as pl
from jax.experimental.pallas import tpu as pltpu
from jax.experimental.pallas import tpu_sc as plsc

NC = 2
NS = 16
L = 16
NW = NC * NS

B = 16384
D = 32
V = 1000000
BPW = B // NW
SE = 4
NSUB = BPW // SE
HT = D // 4

_mesh = plsc.VectorSubcoreMesh(core_axis_name="c", subcore_axis_name="s")


@functools.partial(
    pl.kernel,
    out_type=jax.ShapeDtypeStruct((B,), jnp.float32),
    mesh=_mesh,
    compiler_params=pltpu.CompilerParams(needs_layout_passes=False,
                                         use_tc_tiling_on_sc=True),
    scratch_types=[
        pltpu.VMEM((BPW + L,), jnp.int32),
        pltpu.VMEM((BPW + L,), jnp.int32),
        pltpu.VMEM((3, SE, 4, 8, 128), jnp.float32),
        pltpu.VMEM((3, SE, 4, 8, 128), jnp.float32),
        pltpu.VMEM((L,), jnp.float32),
        pltpu.VMEM((BPW + L,), jnp.float32),
        pltpu.SemaphoreType.DMA,
        pltpu.SemaphoreType.DMA,
        pltpu.SemaphoreType.DMA,
        pltpu.SemaphoreType.DMA,
        pltpu.SemaphoreType.DMA,
        pltpu.SemaphoreType.DMA,
    ],
)
def _mf_sc_kernel(uidx_hbm, iidx_hbm, utab3_hbm, itab3_hbm, out_hbm,
                  uidx_v, iidx_v, ublk_v, vblk_v, fold_v, out_v,
                  usem0, usem1, usem2, vsem0, vsem1, vsem2):
    sid = lax.axis_index("s")
    cid = lax.axis_index("c")
    wid = sid * NC + cid
    base = wid * BPW

    pltpu.sync_copy(uidx_hbm.at[pl.ds(base, BPW)], uidx_v.at[pl.ds(0, BPW)])
    pltpu.sync_copy(iidx_hbm.at[pl.ds(base, BPW)], iidx_v.at[pl.ds(0, BPW)])

    lane = lax.iota(jnp.int32, L)
    lo4 = lax.bitwise_and(lane, 3)
    quad = lax.shift_right_logical(lane, 2)
    outmask = quad == 0

    def issue(tab_hbm, idx_ref, blk, s, par, sem):
        def body(e, c):
            vec = idx_ref[pl.ds(s * SE + e, L)]
            o0 = pl.multiple_of(lax.bitwise_and(vec[0], jnp.int32(~127)), 128)
            pltpu.async_copy(tab_hbm.at[:, :, pl.ds(o0, 128)],
                             blk.at[par, e], sem)
            return c
        lax.fori_loop(0, SE, body, 0)

    def drain(tab_hbm, blk, sem):
        tmpl = pltpu.make_async_copy(tab_hbm.at[:, :, pl.ds(0, 128)],
                                     blk.at[0, 0], sem)
        for _ in range(SE):
            tmpl.wait()

    issue(utab3_hbm, uidx_v, ublk_v, 0, 0, usem0)
    issue(itab3_hbm, iidx_v, vblk_v, 0, 0, vsem0)
    issue(utab3_hbm, uidx_v, ublk_v, 1, 1, usem1)
    issue(itab3_hbm, iidx_v, vblk_v, 1, 1, vsem1)
    issue(utab3_hbm, uidx_v, ublk_v, 2, 2, usem2)
    issue(itab3_hbm, iidx_v, vblk_v, 2, 2, vsem2)

    def sub_body(s, carry):
        e0 = s * SE
        par = lax.rem(s, 3)
        parv = jnp.zeros((L,), jnp.int32) + par

        @pl.when(par == 0)
        def _():
            drain(utab3_hbm, ublk_v, usem0)
            drain(itab3_hbm, vblk_v, vsem0)

        @pl.when(par == 1)
        def _():
            drain(utab3_hbm, ublk_v, usem1)
            drain(itab3_hbm, vblk_v, vsem1)

        @pl.when(par == 2)
        def _():
            drain(utab3_hbm, ublk_v, usem2)
            drain(itab3_hbm, vblk_v, vsem2)

        up = lax.bitwise_and(plsc.load_gather(uidx_v, [e0 + lo4]), jnp.int32(127))
        ip = lax.bitwise_and(plsc.load_gather(iidx_v, [e0 + lo4]), jnp.int32(127))
        acc = jnp.zeros((L,), jnp.float32)
        for t in range(HT):
            j0 = 4 * t
            fb = jnp.full((L,), j0 // 8, jnp.int32)
            fr = jnp.full((L,), j0 % 8, jnp.int32) + quad
            uj = plsc.load_gather(ublk_v, [parv, lo4, fb, fr, up])
            vj = plsc.load_gather(vblk_v, [parv, lo4, fb, fr, ip])
            acc = acc + uj * vj

        fold_v[...] = acc
        acc2 = acc + plsc.load_gather(fold_v, [lax.bitwise_xor(lane, 8)])
        fold_v[...] = acc2
        tot = acc2 + plsc.load_gather(fold_v, [lax.bitwise_xor(lane, 4)])
        plsc.store_compressed(out_v.at[pl.ds(e0, L)], tot, mask=outmask)

        @pl.when((s + 3 < NSUB) & (par == 0))
        def _():
            issue(utab3_hbm, uidx_v, ublk_v, s + 3, 0, usem0)
            issue(itab3_hbm, iidx_v, vblk_v, s + 3, 0, vsem0)

        @pl.when((s + 3 < NSUB) & (par == 1))
        def _():
            issue(utab3_hbm, uidx_v, ublk_v, s + 3, 1, usem1)
            issue(itab3_hbm, iidx_v, vblk_v, s + 3, 1, vsem1)

        @pl.when((s + 3 < NSUB) & (par == 2))
        def _():
            issue(utab3_hbm, uidx_v, ublk_v, s + 3, 2, usem2)
            issue(itab3_hbm, iidx_v, vblk_v, s + 3, 2, vsem2)
        return carry

    lax.fori_loop(0, NSUB, sub_body, 0)
    pltpu.sync_copy(out_v.at[pl.ds(0, BPW)], out_hbm.at[pl.ds(base, BPW)])


def kernel(user_idx, item_idx, user_table, item_table):
    ut3 = user_table.T.reshape(4, 8, V)
    it3 = item_table.T.reshape(4, 8, V)
    return _mf_sc_kernel(user_idx.astype(jnp.int32), item_idx.astype(jnp.int32),
                         ut3, it3)


# speedup vs baseline: 1.2274x; 1.0213x over previous
import functools

import jax
import jax.numpy as jnp
from jax import lax
from jax.experimental import pallas as pl
from jax.experimental.pallas import tpu as pltpu
from jax.experimental.pallas import tpu_sc as plsc

NC = 2
NS = 16
L = 16
NW = NC * NS

B = 16384
D = 32
V = 1000000
BPW = B // NW
SE = 4
NSUB = BPW // SE
HT = D // 4

_mesh = plsc.VectorSubcoreMesh(core_axis_name="c", subcore_axis_name="s")


@functools.partial(
    pl.kernel,
    out_type=jax.ShapeDtypeStruct((B,), jnp.float32),
    mesh=_mesh,
    compiler_params=pltpu.CompilerParams(needs_layout_passes=False,
                                         use_tc_tiling_on_sc=True),
    scratch_types=[
        pltpu.VMEM((BPW + L,), jnp.int32),
        pltpu.VMEM((BPW + L,), jnp.int32),
        pltpu.VMEM((4, SE, 4, 8, 128), jnp.float32),
        pltpu.VMEM((3, SE, 4, 8, 128), jnp.float32),
        pltpu.VMEM((L,), jnp.float32),
        pltpu.VMEM((BPW + L,), jnp.float32),
        pltpu.SemaphoreType.DMA,
        pltpu.SemaphoreType.DMA,
        pltpu.SemaphoreType.DMA,
        pltpu.SemaphoreType.DMA,
        pltpu.SemaphoreType.DMA,
        pltpu.SemaphoreType.DMA,
        pltpu.SemaphoreType.DMA,
    ],
)
def _mf_sc_kernel(uidx_hbm, iidx_hbm, utab3_hbm, itab3_hbm, out_hbm,
                  uidx_v, iidx_v, ublk_v, vblk_v, fold_v, out_v,
                  usem0, usem1, usem2, usem3, vsem0, vsem1, vsem2):
    sid = lax.axis_index("s")
    cid = lax.axis_index("c")
    wid = sid * NC + cid
    base = wid * BPW

    pltpu.sync_copy(uidx_hbm.at[pl.ds(base, BPW)], uidx_v.at[pl.ds(0, BPW)])
    pltpu.sync_copy(iidx_hbm.at[pl.ds(base, BPW)], iidx_v.at[pl.ds(0, BPW)])

    lane = lax.iota(jnp.int32, L)
    lo4 = lax.bitwise_and(lane, 3)
    quad = lax.shift_right_logical(lane, 2)
    outmask = quad == 0

    def issue(tab_hbm, idx_ref, blk, s, par, sem):
        def body(e, c):
            vec = idx_ref[pl.ds(s * SE + e, L)]
            o0 = pl.multiple_of(lax.bitwise_and(vec[0], jnp.int32(~127)), 128)
            pltpu.async_copy(tab_hbm.at[:, :, pl.ds(o0, 128)],
                             blk.at[par, e], sem)
            return c
        lax.fori_loop(0, SE, body, 0)

    def drain(tab_hbm, blk, sem):
        tmpl = pltpu.make_async_copy(tab_hbm.at[:, :, pl.ds(0, 128)],
                                     blk.at[0, 0], sem)
        for _ in range(SE):
            tmpl.wait()

    issue(utab3_hbm, uidx_v, ublk_v, 0, 0, usem0)
    issue(itab3_hbm, iidx_v, vblk_v, 0, 0, vsem0)
    issue(utab3_hbm, uidx_v, ublk_v, 1, 1, usem1)
    issue(itab3_hbm, iidx_v, vblk_v, 1, 1, vsem1)
    issue(utab3_hbm, uidx_v, ublk_v, 2, 2, usem2)
    issue(itab3_hbm, iidx_v, vblk_v, 2, 2, vsem2)
    issue(utab3_hbm, uidx_v, ublk_v, 3, 3, usem3)

    def sub_body(s, carry):
        e0 = s * SE
        paru = lax.bitwise_and(s, 3)
        parv_i = lax.rem(s, 3)
        parvu = jnp.zeros((L,), jnp.int32) + paru
        parvv = jnp.zeros((L,), jnp.int32) + parv_i

        @pl.when(paru == 0)
        def _():
            drain(utab3_hbm, ublk_v, usem0)

        @pl.when(paru == 1)
        def _():
            drain(utab3_hbm, ublk_v, usem1)

        @pl.when(paru == 2)
        def _():
            drain(utab3_hbm, ublk_v, usem2)

        @pl.when(paru == 3)
        def _():
            drain(utab3_hbm, ublk_v, usem3)

        @pl.when(parv_i == 0)
        def _():
            drain(itab3_hbm, vblk_v, vsem0)

        @pl.when(parv_i == 1)
        def _():
            drain(itab3_hbm, vblk_v, vsem1)

        @pl.when(parv_i == 2)
        def _():
            drain(itab3_hbm, vblk_v, vsem2)

        up = lax.bitwise_and(plsc.load_gather(uidx_v, [e0 + lo4]), jnp.int32(127))
        ip = lax.bitwise_and(plsc.load_gather(iidx_v, [e0 + lo4]), jnp.int32(127))
        acc = jnp.zeros((L,), jnp.float32)
        for t in range(HT):
            j0 = 4 * t
            fb = jnp.full((L,), j0 // 8, jnp.int32)
            fr = jnp.full((L,), j0 % 8, jnp.int32) + quad
            uj = plsc.load_gather(ublk_v, [parvu, lo4, fb, fr, up])
            vj = plsc.load_gather(vblk_v, [parvv, lo4, fb, fr, ip])
            acc = acc + uj * vj

        fold_v[...] = acc
        acc2 = acc + plsc.load_gather(fold_v, [lax.bitwise_xor(lane, 8)])
        fold_v[...] = acc2
        tot = acc2 + plsc.load_gather(fold_v, [lax.bitwise_xor(lane, 4)])
        plsc.store_compressed(out_v.at[pl.ds(e0, L)], tot, mask=outmask)

        @pl.when((s + 4 < NSUB) & (paru == 0))
        def _():
            issue(utab3_hbm, uidx_v, ublk_v, s + 4, 0, usem0)

        @pl.when((s + 4 < NSUB) & (paru == 1))
        def _():
            issue(utab3_hbm, uidx_v, ublk_v, s + 4, 1, usem1)

        @pl.when((s + 4 < NSUB) & (paru == 2))
        def _():
            issue(utab3_hbm, uidx_v, ublk_v, s + 4, 2, usem2)

        @pl.when((s + 4 < NSUB) & (paru == 3))
        def _():
            issue(utab3_hbm, uidx_v, ublk_v, s + 4, 3, usem3)

        @pl.when((s + 3 < NSUB) & (parv_i == 0))
        def _():
            issue(itab3_hbm, iidx_v, vblk_v, s + 3, 0, vsem0)

        @pl.when((s + 3 < NSUB) & (parv_i == 1))
        def _():
            issue(itab3_hbm, iidx_v, vblk_v, s + 3, 1, vsem1)

        @pl.when((s + 3 < NSUB) & (parv_i == 2))
        def _():
            issue(itab3_hbm, iidx_v, vblk_v, s + 3, 2, vsem2)
        return carry

    lax.fori_loop(0, NSUB, sub_body, 0)
    pltpu.sync_copy(out_v.at[pl.ds(0, BPW)], out_hbm.at[pl.ds(base, BPW)])


def kernel(user_idx, item_idx, user_table, item_table):
    ut3 = user_table.T.reshape(4, 8, V)
    it3 = item_table.T.reshape(4, 8, V)
    return _mf_sc_kernel(user_idx.astype(jnp.int32), item_idx.astype(jnp.int32),
                         ut3, it3)
